# baseline (device time: 124158 ns/iter reference)
import jax
import jax.numpy as jnp
from jax import lax
from jax.experimental import pallas as pl
from jax.experimental.pallas import tpu as pltpu

M, N = 4096, 2048
C = 8
R = M // C


def kernel(x, pi):
    x2 = x.reshape(M, N)

    def body(x_ref, pi_ref, out_ref, xin, xout, qs, qr, ss, sr,
             xin_sem, xout_sem, q_send, q_recv, s_send, s_recv):
        my_x = lax.axis_index("x")
        my_y = lax.axis_index("y")
        my_z = lax.axis_index("z")
        dst_x = pi_ref[my_x]

        barrier = pltpu.get_barrier_semaphore()
        pl.semaphore_signal(
            barrier,
            inc=1,
            device_id=(1 - my_x, my_y, my_z),
            device_id_type=pl.DeviceIdType.MESH,
        )
        pl.semaphore_wait(barrier, 1)

        @pl.when(dst_x != my_x)
        def _swap():
            rows = lambda c: pl.ds(c * R, R)
            row1 = lambda c: pl.ds(c, 1)

            def in_copy(c):
                return pltpu.make_async_copy(
                    x_ref.at[rows(c), :], xin.at[c % 2], xin_sem.at[c % 2]
                )

            rdma_q = []
            rdma_s = []
            in_copy(0).start()
            for c in range(C):
                if c + 1 < C:
                    in_copy(c + 1).start()
                in_copy(c).wait()
                xc = xin[c % 2]
                amax = jnp.max(jnp.abs(xc), axis=0, keepdims=True)
                amax = jnp.maximum(amax, 1e-30)
                ss[row1(c), :] = amax * (1.0 / 127.0)
                qs[rows(c), :] = jnp.round(xc * (127.0 / amax)).astype(jnp.int8)
                rq = pltpu.make_async_remote_copy(
                    src_ref=qs.at[rows(c), :],
                    dst_ref=qr.at[rows(c), :],
                    send_sem=q_send.at[c],
                    recv_sem=q_recv.at[c],
                    device_id=(dst_x, my_y, my_z),
                    device_id_type=pl.DeviceIdType.MESH,
                )
                rs = pltpu.make_async_remote_copy(
                    src_ref=ss.at[row1(c), :],
                    dst_ref=sr.at[row1(c), :],
                    send_sem=s_send.at[c],
                    recv_sem=s_recv.at[c],
                    device_id=(dst_x, my_y, my_z),
                    device_id_type=pl.DeviceIdType.MESH,
                )
                rq.start()
                rs.start()
                rdma_q.append(rq)
                rdma_s.append(rs)

            out_copies = []
            for c in range(C):
                rdma_q[c].wait_recv()
                rdma_s[c].wait_recv()
                if c >= 2:
                    out_copies[c - 2].wait()
                xout[c % 2] = qr[rows(c), :].astype(jnp.float32) * sr[row1(c), :]
                oc = pltpu.make_async_copy(
                    xout.at[c % 2], out_ref.at[rows(c), :], xout_sem.at[c % 2]
                )
                oc.start()
                out_copies.append(oc)
            out_copies[C - 2].wait()
            out_copies[C - 1].wait()
            for c in range(C):
                rdma_q[c].wait_send()
                rdma_s[c].wait_send()

        @pl.when(dst_x == my_x)
        def _identity():
            cp = pltpu.make_async_copy(x_ref, out_ref, xin_sem.at[0])
            cp.start()
            cp.wait()

    out2 = pl.pallas_call(
        body,
        out_shape=jax.ShapeDtypeStruct((M, N), jnp.float32),
        in_specs=[
            pl.BlockSpec(memory_space=pl.ANY),
            pl.BlockSpec(memory_space=pltpu.SMEM),
        ],
        out_specs=pl.BlockSpec(memory_space=pl.ANY),
        scratch_shapes=[
            pltpu.VMEM((2, R, N), jnp.float32),
            pltpu.VMEM((2, R, N), jnp.float32),
            pltpu.VMEM((M, N), jnp.int8),
            pltpu.VMEM((M, N), jnp.int8),
            pltpu.VMEM((C, N), jnp.float32),
            pltpu.VMEM((C, N), jnp.float32),
            pltpu.SemaphoreType.DMA((2,)),
            pltpu.SemaphoreType.DMA((2,)),
            pltpu.SemaphoreType.DMA((C,)),
            pltpu.SemaphoreType.DMA((C,)),
            pltpu.SemaphoreType.DMA((C,)),
            pltpu.SemaphoreType.DMA((C,)),
        ],
        compiler_params=pltpu.CompilerParams(
            collective_id=0, vmem_limit_bytes=48 * 1024 * 1024
        ),
    )(x2, pi)
    return out2.reshape(1, M, N)
